# Initial kernel scaffold; baseline (speedup 1.0000x reference)
import jax
import jax.numpy as jnp
from jax.experimental import pallas as pl


def kernel(images, feat0, feat1, feat2, feat3, feat4, conv_w, conv_b, cls_w, cls_b, bbox_w, bbox_b):
    raise NotImplementedError("write your pallas kernel here")


# speedup vs baseline: 12.5742x; 12.5742x over previous
import math
import functools

import jax
import jax.numpy as jnp
from jax import lax
from jax.experimental import pallas as pl
from jax.experimental.pallas import tpu as pltpu

IMG = 512
FEAT_SIZES = [(128, 128), (64, 64), (32, 32), (16, 16), (8, 8)]
SIZES = [32, 64, 128, 256, 512]
PRE_NMS_TOP_N = 1000
POST_NMS_TOP_N = 1000
NMS_THRESH = 0.7
MIN_SIZE = 1e-3
CLIP = math.log(1000.0 / 16.0)
HB = 8
NTOT = 4096
NBLK = NTOT // 128


def _anchors_for_level(lvl):
    import numpy as np
    s = SIZES[lvl]
    fh, fw = FEAT_SIZES[lvl]
    ratios = np.array([0.5, 1.0, 2.0])
    h_r = np.sqrt(ratios)
    w_r = 1.0 / h_r
    ws = w_r * s
    hs = h_r * s
    base = np.round(np.stack([-ws, -hs, ws, hs], axis=1) / 2.0)
    sth, stw = IMG // fh, IMG // fw
    sy = np.arange(fh) * sth
    sx = np.arange(fw) * stw
    yy, xx = np.meshgrid(sy, sx, indexing='ij')
    shifts = np.stack([xx.ravel(), yy.ravel(), xx.ravel(), yy.ravel()], axis=1)
    return (shifts[:, None, :] + base[None, :, :]).reshape(-1, 4).astype(np.float32)


def _conv_level_kernel(x0_ref, x1_ref, x2_ref, w_ref, cw_ref, cb_ref, anch_ref,
                       obj_ref, box_ref, *, W):
    xs = [x0_ref[0], x1_ref[0], x2_ref[0]]
    M = W * HB
    acc = jnp.zeros((M, 256), jnp.float32)
    for ky in range(3):
        for kx in range(3):
            xk = xs[ky][kx:kx + W].reshape(M, 256)
            acc += lax.dot_general(
                xk, w_ref[ky, kx], (((1,), (0,)), ((), ())),
                preferred_element_type=jnp.float32,
                precision=lax.Precision.DEFAULT)
    t = jnp.maximum(acc, 0.0)
    lg = lax.dot_general(t, cw_ref[...], (((1,), (0,)), ((), ())),
                         preferred_element_type=jnp.float32,
                         precision=lax.Precision.DEFAULT)
    lg = lg + cb_ref[0:1, :]
    dx = lg[:, 0:3]
    dy = lg[:, 3:6]
    dw = jnp.minimum(lg[:, 6:9], CLIP)
    dh = jnp.minimum(lg[:, 9:12], CLIP)
    ax0 = anch_ref[:, 0:3]
    ay0 = anch_ref[:, 3:6]
    ax1 = anch_ref[:, 6:9]
    ay1 = anch_ref[:, 9:12]
    aw = ax1 - ax0
    ah = ay1 - ay0
    acx = ax0 + 0.5 * aw
    acy = ay0 + 0.5 * ah
    pcx = dx * aw + acx
    pcy = dy * ah + acy
    pw = jnp.exp(dw) * aw
    ph = jnp.exp(dh) * ah
    fimg = float(IMG)
    bx0 = jnp.clip(pcx - 0.5 * pw, 0.0, fimg)
    by0 = jnp.clip(pcy - 0.5 * ph, 0.0, fimg)
    bx1 = jnp.clip(pcx + 0.5 * pw, 0.0, fimg)
    by1 = jnp.clip(pcy + 0.5 * ph, 0.0, fimg)
    zpad_o = jnp.zeros((M, 125), jnp.float32)
    obj_ref[0] = jnp.concatenate([lg[:, 12:15], zpad_o], axis=1)
    zpad_b = jnp.zeros((M, 116), jnp.float32)
    box_ref[0] = jnp.concatenate([bx0, by0, bx1, by1, zpad_b], axis=1)


def _run_conv_level(feat, w9, cw, cb, anchB, W, H):
    x = jnp.transpose(feat, (0, 2, 3, 1))
    xp = jnp.pad(x, ((0, 0), (1, 1), (1, 1), (0, 0)))
    xs = [jnp.transpose(xp[:, ky:ky + H], (0, 2, 1, 3)) for ky in range(3)]
    nblk = H // HB
    M = W * HB
    grid = (2, nblk)
    xspec = pl.BlockSpec((1, W + 2, HB, 256), lambda n, i: (n, 0, i, 0))
    obj, box = pl.pallas_call(
        functools.partial(_conv_level_kernel, W=W),
        grid=grid,
        in_specs=[
            xspec, xspec, xspec,
            pl.BlockSpec((3, 3, 256, 256), lambda n, i: (0, 0, 0, 0)),
            pl.BlockSpec((256, 16), lambda n, i: (0, 0)),
            pl.BlockSpec((8, 16), lambda n, i: (0, 0)),
            pl.BlockSpec((M, 16), lambda n, i: (i, 0)),
        ],
        out_specs=[
            pl.BlockSpec((1, M, 128), lambda n, i: (n, i, 0)),
            pl.BlockSpec((1, M, 128), lambda n, i: (n, i, 0)),
        ],
        out_shape=[
            jax.ShapeDtypeStruct((2, H * W, 128), jnp.float32),
            jax.ShapeDtypeStruct((2, H * W, 128), jnp.float32),
        ],
        compiler_params=pltpu.CompilerParams(
            dimension_semantics=("parallel", "arbitrary")),
    )(xs[0], xs[1], xs[2], w9, cw, cb, anchB)
    return obj, box


def _nms_kernel(sba_ref, sbt_ref, ss_ref, out_ref, m_sc, sup_sc, keep_sc):
    bi = pl.program_id(1)
    base = pl.multiple_of(bi * 128, 128)
    lane128 = lax.broadcasted_iota(jnp.int32, (1, 128), 1)

    @pl.when(bi == 0)
    def _init():
        sup_sc[...] = jnp.zeros((1, NTOT), jnp.float32)
        keep_sc[...] = jnp.zeros((1, NTOT), jnp.float32)

    rows = sba_ref[0, pl.ds(base, 128), :]
    x0r, y0r, x1r, y1r = (rows[:, 4:5], rows[:, 5:6], rows[:, 6:7], rows[:, 7:8])
    x0c = sbt_ref[0, 4:5, pl.ds(base, 128)]
    y0c = sbt_ref[0, 5:6, pl.ds(base, 128)]
    x1c = sbt_ref[0, 6:7, pl.ds(base, 128)]
    y1c = sbt_ref[0, 7:8, pl.ds(base, 128)]
    area_r = (x1r - x0r) * (y1r - y0r)
    area_c = (x1c - x0c) * (y1c - y0c)
    ltx = jnp.maximum(x0r, x0c)
    lty = jnp.maximum(y0r, y0c)
    rbx = jnp.minimum(x1r, x1c)
    rby = jnp.minimum(y1r, y1c)
    wx = jnp.clip(rbx - ltx, 0.0, None)
    wy = jnp.clip(rby - lty, 0.0, None)
    inter = wx * wy
    iou = inter / (area_r + area_c - inter + 1e-9)
    m_sc[...] = (iou > NMS_THRESH).astype(jnp.float32)

    cx0 = sbt_ref[0, 0:1, pl.ds(base, 128)]
    cy0 = sbt_ref[0, 1:2, pl.ds(base, 128)]
    cx1 = sbt_ref[0, 2:3, pl.ds(base, 128)]
    cy1 = sbt_ref[0, 3:4, pl.ds(base, 128)]
    valid_b = jnp.logical_and(cx1 - cx0 >= MIN_SIZE,
                              cy1 - cy0 >= MIN_SIZE).astype(jnp.float32)

    sup_b0 = sup_sc[0:1, pl.ds(base, 128)]

    def body(i, carry):
        sup_b, keep_b = carry
        mask_i = (lane128 == i).astype(jnp.float32)
        k_i = jnp.sum(mask_i * valid_b * (1.0 - sup_b))
        keep_b = keep_b + mask_i * k_i
        row_i = lax.dot_general(mask_i, m_sc[...], (((1,), (0,)), ((), ())),
                                preferred_element_type=jnp.float32,
                                precision=lax.Precision.HIGHEST)
        gt_i = (lane128 > i).astype(jnp.float32)
        sup_b = jnp.maximum(sup_b, row_i * (k_i * gt_i))
        return sup_b, keep_b

    sup_b, keep_b = lax.fori_loop(0, 128, body, (sup_b0, jnp.zeros((1, 128), jnp.float32)))
    keep_sc[0:1, pl.ds(base, 128)] = keep_b

    def cross(c, _):
        @pl.when(c * 1024 + 1024 > base + 128)
        def _():
            cbase = pl.multiple_of(c * 1024, 1024)
            ax0 = sbt_ref[0, 4:5, pl.ds(cbase, 1024)]
            ay0 = sbt_ref[0, 5:6, pl.ds(cbase, 1024)]
            ax1 = sbt_ref[0, 6:7, pl.ds(cbase, 1024)]
            ay1 = sbt_ref[0, 7:8, pl.ds(cbase, 1024)]
            a_c = (ax1 - ax0) * (ay1 - ay0)
            ltx2 = jnp.maximum(x0r, ax0)
            lty2 = jnp.maximum(y0r, ay0)
            rbx2 = jnp.minimum(x1r, ax1)
            rby2 = jnp.minimum(y1r, ay1)
            wx2 = jnp.clip(rbx2 - ltx2, 0.0, None)
            wy2 = jnp.clip(rby2 - lty2, 0.0, None)
            inter2 = wx2 * wy2
            iou2 = inter2 / (area_r + a_c - inter2 + 1e-9)
            hi = (iou2 > NMS_THRESH).astype(jnp.float32)
            supadd = lax.dot_general(keep_b, hi, (((1,), (0,)), ((), ())),
                                     preferred_element_type=jnp.float32,
                                     precision=lax.Precision.HIGHEST)
            lane1024 = lax.broadcasted_iota(jnp.int32, (1, 1024), 1) + cbase
            late = (lane1024 > base + 127).astype(jnp.float32)
            newsup = jnp.minimum(supadd, 1.0) * late
            old = sup_sc[0:1, pl.ds(cbase, 1024)]
            sup_sc[0:1, pl.ds(cbase, 1024)] = jnp.maximum(old, newsup)
        return 0

    lax.fori_loop(0, NTOT // 1024, cross, 0)

    @pl.when(bi == NBLK - 1)
    def _finalize():
        keep_rows = jnp.concatenate(
            [keep_sc[0:1, j * 128:(j + 1) * 128] for j in range(NBLK)], axis=0)
        l128 = lax.broadcasted_iota(jnp.int32, (128, 128), 0)
        u128 = lax.broadcasted_iota(jnp.int32, (128, 128), 1)
        triu = (l128 <= u128).astype(jnp.float32)
        incl = lax.dot_general(keep_rows, triu, (((1,), (0,)), ((), ())),
                               preferred_element_type=jnp.float32,
                               precision=lax.Precision.HIGHEST)
        row_tot = incl[:, 127:128]
        ls = lax.broadcasted_iota(jnp.int32, (NBLK, NBLK), 0)
        us = lax.broadcasted_iota(jnp.int32, (NBLK, NBLK), 1)
        tril_s = (us < ls).astype(jnp.float32)
        row_off = lax.dot_general(tril_s, row_tot, (((1,), (0,)), ((), ())),
                                  preferred_element_type=jnp.float32,
                                  precision=lax.Precision.HIGHEST)
        pos = row_off + incl - 1.0
        srow = lax.broadcasted_iota(jnp.int32, (1024, 128), 0).astype(jnp.float32)
        acc = jnp.zeros((1024, 8), jnp.float32)
        for b2 in range(NBLK):
            pos_b = pos[b2:b2 + 1, :]
            keep_b2 = keep_rows[b2:b2 + 1, :]
            oh = (srow == pos_b).astype(jnp.float32) * keep_b2
            pay_b = jnp.concatenate(
                [sba_ref[0, b2 * 128:(b2 + 1) * 128, 0:4],
                 ss_ref[0, b2 * 128:(b2 + 1) * 128, :],
                 jnp.zeros((128, 3), jnp.float32)], axis=1)
            acc += lax.dot_general(oh, pay_b, (((1,), (0,)), ((), ())),
                                   preferred_element_type=jnp.float32,
                                   precision=lax.Precision.HIGHEST)
        out_ref[0] = acc


def _run_nms(sb_all, sbT, ss_col):
    out = pl.pallas_call(
        _nms_kernel,
        grid=(2, NBLK),
        in_specs=[
            pl.BlockSpec((1, NTOT, 8), lambda n, b: (n, 0, 0)),
            pl.BlockSpec((1, 8, NTOT), lambda n, b: (n, 0, 0)),
            pl.BlockSpec((1, NTOT, 1), lambda n, b: (n, 0, 0)),
        ],
        out_specs=pl.BlockSpec((1, 1024, 8), lambda n, b: (n, 0, 0)),
        out_shape=jax.ShapeDtypeStruct((2, 1024, 8), jnp.float32),
        scratch_shapes=[
            pltpu.VMEM((128, 128), jnp.float32),
            pltpu.VMEM((1, NTOT), jnp.float32),
            pltpu.VMEM((1, NTOT), jnp.float32),
        ],
        compiler_params=pltpu.CompilerParams(
            dimension_semantics=("arbitrary", "arbitrary")),
    )(sb_all, sbT, ss_col)
    return out


def kernel(images, feat0, feat1, feat2, feat3, feat4,
           conv_w, conv_b, cls_w, cls_b, bbox_w, bbox_b):
    feats = [feat0, feat1, feat2, feat3, feat4]
    w9 = jnp.transpose(conv_w, (2, 3, 1, 0))
    bw = bbox_w[:, :, 0, 0]
    bw = bw.reshape(3, 4, 256)
    bw = jnp.transpose(bw, (1, 0, 2))
    bw = bw.reshape(12, 256)
    cwm = jnp.concatenate([bw, cls_w[:, :, 0, 0],
                           jnp.zeros((1, 256), jnp.float32)], axis=0)
    cw = jnp.transpose(cwm, (1, 0))
    bb = bbox_b.reshape(3, 4).T.reshape(12)
    cb_row = jnp.concatenate([bb, cls_b, jnp.zeros((1,), jnp.float32)])
    cb = jnp.zeros((8, 16), jnp.float32).at[0].set(cb_row)

    sc_l, bx_l, lv_l = [], [], []
    for lvl, feat in enumerate(feats):
        H, W = FEAT_SIZES[lvl]
        anch = _anchors_for_level(lvl)
        import numpy as np
        a4 = anch.reshape(H, W, 3, 4)
        a4 = a4.transpose(0, 1, 3, 2).reshape(H, W, 12)
        a4 = a4.reshape(H // HB, HB, W, 12).transpose(0, 2, 1, 3)
        a4 = a4.reshape(H * W, 12)
        anchB = jnp.asarray(np.concatenate(
            [a4, np.zeros((H * W, 4), np.float32)], axis=1))
        obj, box = _run_conv_level(feat, w9, cw, cb, anchB, W, H)
        objf = obj[:, :, 0:3].reshape(2, H * W * 3)
        b4 = box[:, :, 0:12].reshape(2, H * W, 4, 3)
        b4 = jnp.transpose(b4, (0, 1, 3, 2)).reshape(2, H * W * 3, 4)
        k = min(PRE_NMS_TOP_N, objf.shape[1])
        top_s, top_i = lax.top_k(objf, k)
        top_b = jnp.take_along_axis(b4, top_i[..., None], axis=1)
        sc_l.append(top_s)
        bx_l.append(top_b)
        lv_l.append(jnp.full((2, k), float(lvl), dtype=jnp.float32))

    scores = jax.nn.sigmoid(jnp.concatenate(sc_l, axis=1))
    boxes = jnp.concatenate(bx_l, axis=1)
    levels = jnp.concatenate(lv_l, axis=1)
    n_real = scores.shape[1]
    pad = NTOT - n_real
    scores_p = jnp.pad(scores, ((0, 0), (0, pad)))
    boxes_p = jnp.pad(boxes, ((0, 0), (0, pad), (0, 0)))
    levels_p = jnp.pad(levels, ((0, 0), (0, pad)))
    boxes_off = boxes_p + (levels_p * (IMG + 1.0))[..., None]

    order = jnp.argsort(-scores_p, axis=1)
    sb = jnp.take_along_axis(boxes_p, order[..., None], axis=1)
    sboff = jnp.take_along_axis(boxes_off, order[..., None], axis=1)
    ss = jnp.take_along_axis(scores_p, order, axis=1)
    sb_all = jnp.concatenate([sb, sboff], axis=2)
    sbT = jnp.transpose(sb_all, (0, 2, 1))
    ss_col = ss[..., None]

    out = _run_nms(sb_all, sbT, ss_col)
    out_b = out[:, :POST_NMS_TOP_N, 0:4]
    out_s = out[:, :POST_NMS_TOP_N, 4]
    return out_b, out_s
